# baseline (speedup 1.0000x reference)
import functools

import jax
import jax.numpy as jnp
from jax import lax
from jax.experimental import pallas as pl
from jax.experimental.pallas import tpu as pltpu
from jax.experimental.pallas import tpu_sc as plsc

FIELD = 38461
NFIELD = 26
EMBED = 16
BATCH = 16384
FLAT = BATCH * NFIELD
NC, NS, LANES = 2, 16, 16
NW = NC * NS
PER_W = FLAT // NW
GROUP = 128
NG = PER_W // GROUP


def _body(x_hbm, table_hbm, out_hbm, idx_v, buf_v, sem):
    wid = lax.axis_index("s") * NC + lax.axis_index("c")
    row0 = wid * NG
    base = wid * PER_W

    pltpu.sync_copy(x_hbm.at[pl.ds(row0, NG), :], idx_v)

    lanes = lax.iota(jnp.int32, LANES)

    @pl.loop(0, NG)
    def _offsets(r):
        for c in range(GROUP // LANES):
            p0 = r * GROUP + c * LANES
            offs = lax.rem(p0 + lanes, NFIELD) * FIELD
            idx_v[r, pl.ds(c * LANES, LANES)] += offs

    @pl.loop(0, NG)
    def _gather(g):
        pltpu.async_copy(table_hbm.at[idx_v.at[g]], buf_v, sem).wait()
        pltpu.sync_copy(buf_v, out_hbm.at[pl.ds(base + g * GROUP, GROUP), :])


def kernel(x, table):
    x2d = x.reshape(FLAT // GROUP, GROUP)
    mesh = plsc.VectorSubcoreMesh(core_axis_name="c", subcore_axis_name="s")
    run = functools.partial(
        pl.kernel,
        mesh=mesh,
        out_type=jax.ShapeDtypeStruct((FLAT, EMBED), jnp.float32),
        scratch_types=[
            pltpu.VMEM((NG, GROUP), jnp.int32),
            pltpu.VMEM((GROUP, EMBED), jnp.float32),
            pltpu.SemaphoreType.DMA,
        ],
        compiler_params=pltpu.CompilerParams(use_tc_tiling_on_sc=False),
    )(_body)
    out = run(x2d, table)
    return out.reshape(BATCH, NFIELD, EMBED)


# speedup vs baseline: 1.0750x; 1.0750x over previous
import functools

import jax
import jax.numpy as jnp
from jax import lax
from jax.experimental import pallas as pl
from jax.experimental.pallas import tpu as pltpu
from jax.experimental.pallas import tpu_sc as plsc

FIELD = 38461
NFIELD = 26
EMBED = 16
BATCH = 16384
FLAT = BATCH * NFIELD
NC, NS, LANES = 2, 16, 16
NW = NC * NS
PER_W = FLAT // NW
GROUP = 128
NG = PER_W // GROUP


NPASS = 4
PASS_G = NG // NPASS
PASS_R = PASS_G * GROUP


def _body(x_hbm, table_hbm, out_hbm, idx_v, buf_a, buf_b, sem_g, sem_o0, sem_o1):
    wid = lax.axis_index("s") * NC + lax.axis_index("c")
    row0 = wid * NG
    base = wid * PER_W

    pltpu.sync_copy(x_hbm.at[pl.ds(row0, NG), :], idx_v)

    lanes = lax.iota(jnp.int32, LANES)

    @pl.loop(0, NG)
    def _offsets(r):
        for c in range(GROUP // LANES):
            p0 = r * GROUP + c * LANES
            offs = lax.rem(p0 + lanes, NFIELD) * FIELD
            idx_v[r, pl.ds(c * LANES, LANES)] += offs

    bufs = (buf_a, buf_b)
    sems = (sem_o0, sem_o1)
    writes = [None, None]
    for p in range(NPASS):
        buf = bufs[p % 2]
        if writes[p % 2] is not None:
            writes[p % 2].wait()

        @pl.loop(0, PASS_G)
        def _fire(g, p=p, buf=buf):
            gg = p * PASS_G + g
            pltpu.async_copy(
                table_hbm.at[idx_v.at[gg]],
                buf.at[pl.ds(g * GROUP, GROUP), :],
                sem_g,
            )

        out_slice = out_hbm.at[pl.ds(base + p * PASS_R, PASS_R), :]
        pltpu.make_async_copy(out_slice, buf, sem_g).wait()
        writes[p % 2] = pltpu.async_copy(buf, out_slice, sems[p % 2])
    writes[0].wait()
    writes[1].wait()


def kernel(x, table):
    x2d = x.reshape(FLAT // GROUP, GROUP)
    mesh = plsc.VectorSubcoreMesh(core_axis_name="c", subcore_axis_name="s")
    run = functools.partial(
        pl.kernel,
        mesh=mesh,
        out_type=jax.ShapeDtypeStruct((FLAT, EMBED), jnp.float32),
        scratch_types=[
            pltpu.VMEM((NG, GROUP), jnp.int32),
            pltpu.VMEM((PASS_R, EMBED), jnp.float32),
            pltpu.VMEM((PASS_R, EMBED), jnp.float32),
            pltpu.SemaphoreType.DMA,
            pltpu.SemaphoreType.DMA,
            pltpu.SemaphoreType.DMA,
        ],
        compiler_params=pltpu.CompilerParams(use_tc_tiling_on_sc=False),
    )(_body)
    out = run(x2d, table)
    return out.reshape(BATCH, NFIELD, EMBED)


# speedup vs baseline: 6.9923x; 6.5042x over previous
import functools

import jax
import jax.numpy as jnp
from jax import lax
from jax.experimental import pallas as pl
from jax.experimental.pallas import tpu as pltpu
from jax.experimental.pallas import tpu_sc as plsc

FIELD = 38461
NFIELD = 26
EMBED = 16
BATCH = 16384
NROWS = FIELD * NFIELD
NC, NS, LANES = 2, 16, 16
F_PER_CORE = NFIELD // NC
WIN = 38656
TAIL_COL = (NROWS // 128) * 128
TAIL_WIN = 38528
TAIL_PAD = 128
NVEC = BATCH // LANES


def _body(x_hbm, tab_hbm, aux_hbm, out_hbm, xrow_v, win_v, out_v):
    c = lax.axis_index("s")
    k = lax.axis_index("c")

    for j in range(F_PER_CORE):
        f = k * F_PER_CORE + j
        pltpu.sync_copy(x_hbm.at[f, :], xrow_v)
        o = f * FIELD
        col0 = pl.multiple_of(o - lax.rem(o, 128), 128)
        pad = o - col0
        if j == F_PER_CORE - 1:
            @pl.when(k == 0)
            def _full():
                pltpu.sync_copy(tab_hbm.at[c, pl.ds(col0, WIN)],
                                win_v.at[pl.ds(0, WIN)])

            @pl.when(k == 1)
            def _tail():
                pltpu.sync_copy(tab_hbm.at[c, pl.ds(col0, TAIL_WIN)],
                                win_v.at[pl.ds(0, TAIL_WIN)])
                pltpu.sync_copy(aux_hbm.at[c, :],
                                win_v.at[pl.ds(TAIL_WIN, TAIL_PAD)])
        else:
            pltpu.sync_copy(tab_hbm.at[c, pl.ds(col0, WIN)],
                            win_v.at[pl.ds(0, WIN)])

        @pl.loop(0, NVEC)
        def _gather(g, pad=pad):
            idx = xrow_v[pl.ds(g * LANES, LANES)] + pad
            out_v[pl.ds(g * LANES, LANES)] = plsc.load_gather(win_v, [idx])

        pltpu.sync_copy(out_v, out_hbm.at[f, c, :])


def kernel(x, table):
    x_t = x.T
    tab_t = table.T
    aux = jnp.pad(table[TAIL_COL:, :].T, ((0, 0), (0, TAIL_PAD - (NROWS - TAIL_COL))))
    mesh = plsc.VectorSubcoreMesh(core_axis_name="c", subcore_axis_name="s")
    run = functools.partial(
        pl.kernel,
        mesh=mesh,
        out_type=jax.ShapeDtypeStruct((NFIELD, EMBED, BATCH), jnp.float32),
        scratch_types=[
            pltpu.VMEM((BATCH,), jnp.int32),
            pltpu.VMEM((WIN,), jnp.float32),
            pltpu.VMEM((BATCH,), jnp.float32),
        ],
        compiler_params=pltpu.CompilerParams(needs_layout_passes=False),
    )(_body)
    out = run(x_t, tab_t, aux)
    return out.transpose(2, 0, 1)
